# baseline (speedup 1.0000x reference)
import functools

import jax
import jax.numpy as jnp
from jax import lax
from jax.experimental import pallas as pl
from jax.experimental.pallas import tpu as pltpu
from jax.experimental.pallas import tpu_sc as plsc

R = 16384
C = 1024
NC, NS = 2, 16
NW = NC * NS
RPW = R // NW
G = 32
NG = RPW // G
L = 16
CH = C // L


_GATHER_DNUMS = lax.GatherDimensionNumbers(
    offset_dims=(), collapsed_slice_dims=(0,), start_index_map=(0,))


def _shuffle(x, idx):
    return lax.gather(
        x, idx[:, None], _GATHER_DNUMS, slice_sizes=(1,),
        mode=lax.GatherScatterMode.PROMISE_IN_BOUNDS)


def _argmax_kernel(sim_hbm, out_hbm, buf, out_v, sem):
    cid = lax.axis_index("c")
    sid = lax.axis_index("s")
    wid = sid * NC + cid
    base_row = wid * RPW
    lane = lax.iota(jnp.int32, L)

    def group_body(g, _):
        row0 = base_row + g * G
        pltpu.sync_copy(sim_hbm.at[pl.ds(row0, G)], buf)

        def tile_body(t, _):
            def row_body(rr, res_vec):
                r = t * L + rr
                bv = buf[r, pl.ds(0, L)]
                bc = jnp.zeros((L,), jnp.int32)
                for i in range(1, CH):
                    v = buf[r, pl.ds(i * L, L)]
                    gt = v > bv
                    bv = jnp.where(gt, v, bv)
                    bc = jnp.where(gt, jnp.full((L,), i, jnp.int32), bc)
                m = bv
                for k in (1, 2, 4, 8):
                    m = jnp.maximum(m, _shuffle(m, (lane + k) & (L - 1)))
                gidx = bc * L + lane
                cand = jnp.where(
                    bv == m, gidx, jnp.full((L,), 2**30, jnp.int32))
                for k in (1, 2, 4, 8):
                    cand = jnp.minimum(
                        cand, _shuffle(cand, (lane + k) & (L - 1)))
                return jnp.where(lane == rr, cand, res_vec)

            res_vec = lax.fori_loop(
                0, L, row_body, jnp.zeros((L,), jnp.int32))
            out_v[pl.ds(g * G + t * L, L)] = res_vec
            return 0

        lax.fori_loop(0, G // L, tile_body, 0)
        return 0

    lax.fori_loop(0, NG, group_body, 0)
    pltpu.sync_copy(out_v, out_hbm.at[pl.ds(base_row, RPW)])


@jax.jit
def _argmax_rows(sim):
    mesh = plsc.VectorSubcoreMesh(core_axis_name="c", subcore_axis_name="s")
    return pl.kernel(
        _argmax_kernel,
        out_type=jax.ShapeDtypeStruct((R,), jnp.int32),
        mesh=mesh,
        scratch_types=[
            pltpu.VMEM((G, C), jnp.float32),
            pltpu.VMEM((RPW,), jnp.int32),
            pltpu.SemaphoreType.DMA,
        ],
    )(sim)


def kernel(distance, kmeans_centers, similarities):
    sim = similarities.reshape(R, C)
    return _argmax_rows(sim).reshape(similarities.shape[:-1])


# speedup vs baseline: 1.6171x; 1.6171x over previous
import functools

import jax
import jax.numpy as jnp
from jax import lax
from jax.experimental import pallas as pl
from jax.experimental.pallas import tpu as pltpu
from jax.experimental.pallas import tpu_sc as plsc

R = 16384
C = 1024
NC, NS = 2, 16
NW = NC * NS
RPW = R // NW
G = 32
NG = RPW // G
L = 16
CH = C // L


_GATHER_DNUMS = lax.GatherDimensionNumbers(
    offset_dims=(), collapsed_slice_dims=(0,), start_index_map=(0,))


def _shuffle(x, idx):
    return lax.gather(
        x, idx[:, None], _GATHER_DNUMS, slice_sizes=(1,),
        mode=lax.GatherScatterMode.PROMISE_IN_BOUNDS)


NACC = 4


def _argmax_kernel(sim_hbm, out_hbm, buf0, buf1, out_v, sem0, sem1):
    cid = lax.axis_index("c")
    sid = lax.axis_index("s")
    wid = sid * NC + cid
    base_row = wid * RPW
    lane = lax.iota(jnp.int32, L)
    big = jnp.full((L,), 2**30, jnp.int32)

    def start(g, buf, sem):
        pltpu.async_copy(sim_hbm.at[pl.ds(base_row + g * G, G)], buf, sem)

    def wait(g, buf, sem):
        pltpu.make_async_copy(
            sim_hbm.at[pl.ds(base_row + g * G, G)], buf, sem).wait()

    def process(g, buf):
        def tile_body(t, _):
            def row_body(rr, res_vec):
                r = t * L + rr
                bv = [buf[r, pl.ds(a * L, L)] for a in range(NACC)]
                bc = [jnp.full((L,), a, jnp.int32) for a in range(NACC)]
                for i in range(NACC, CH):
                    a = i % NACC
                    v = buf[r, pl.ds(i * L, L)]
                    gt = v > bv[a]
                    bv[a] = jnp.where(gt, v, bv[a])
                    bc[a] = jnp.where(gt, jnp.full((L,), i, jnp.int32),
                                      bc[a])

                def merge(p, q):
                    vp, cp = p
                    vq, cq = q
                    take = (vq > vp) | ((vq == vp) & (cq < cp))
                    return (jnp.where(take, vq, vp),
                            jnp.where(take, cq, cp))

                mv, mc = merge(merge((bv[0], bc[0]), (bv[1], bc[1])),
                               merge((bv[2], bc[2]), (bv[3], bc[3])))
                m = mv
                for k in (1, 2, 4, 8):
                    m = jnp.maximum(m, _shuffle(m, (lane + k) & (L - 1)))
                gidx = mc * L + lane
                cand = jnp.where(mv == m, gidx, big)
                for k in (1, 2, 4, 8):
                    cand = jnp.minimum(
                        cand, _shuffle(cand, (lane + k) & (L - 1)))
                return jnp.where(lane == rr, cand, res_vec)

            res_vec = lax.fori_loop(
                0, L, row_body, jnp.zeros((L,), jnp.int32))
            out_v[pl.ds(g * G + t * L, L)] = res_vec
            return 0

        lax.fori_loop(0, G // L, tile_body, 0)

    start(0, buf0, sem0)

    def pair_body(h, _):
        g0 = 2 * h
        wait(g0, buf0, sem0)
        start(g0 + 1, buf1, sem1)
        process(g0, buf0)
        wait(g0 + 1, buf1, sem1)

        @pl.when(g0 + 2 < NG)
        def _():
            start(g0 + 2, buf0, sem0)

        process(g0 + 1, buf1)
        return 0

    lax.fori_loop(0, NG // 2, pair_body, 0)
    pltpu.sync_copy(out_v, out_hbm.at[pl.ds(base_row, RPW)])


@jax.jit
def _argmax_rows(sim):
    mesh = plsc.VectorSubcoreMesh(core_axis_name="c", subcore_axis_name="s")
    return pl.kernel(
        _argmax_kernel,
        out_type=jax.ShapeDtypeStruct((R,), jnp.int32),
        mesh=mesh,
        scratch_types=[
            pltpu.VMEM((G, C), jnp.float32),
            pltpu.VMEM((G, C), jnp.float32),
            pltpu.VMEM((RPW,), jnp.int32),
            pltpu.SemaphoreType.DMA,
            pltpu.SemaphoreType.DMA,
        ],
    )(sim)


def kernel(distance, kmeans_centers, similarities):
    sim = similarities.reshape(R, C)
    return _argmax_rows(sim).reshape(similarities.shape[:-1])
